# Initial kernel scaffold; baseline (speedup 1.0000x reference)
import jax
import jax.numpy as jnp
from jax.experimental import pallas as pl


def kernel(x, edge_index, W1, b1, W2, b2):
    raise NotImplementedError("write your pallas kernel here")


# speedup vs baseline: 3.3133x; 3.3133x over previous
import functools

import jax
import jax.numpy as jnp
from jax import lax
from jax.experimental import pallas as pl
from jax.experimental.pallas import tpu as pltpu
from jax.experimental.pallas import tpu_sc as plsc

N_NODES = 10000
D = 128

NC = 2
NS = 16
NW = NC * NS

CH = 80
SUB = 32
N_SUP = 4
N_CH = SUB * N_SUP
E_T = N_CH * CH
E_PAD = NW * E_T
R_ACC = 10240
R_T = R_ACC // NS
DUMMY = N_NODES


def _agg_body(x_hbm, idx_hbm, out_hbm,
              ibuf_a, ibuf_b, gbuf_a, gbuf_b, acc_sh, sem_a, sem_b, sem_i):
    c = lax.axis_index("c")
    s = lax.axis_index("s")
    wid = s * NC + c

    pltpu.sync_copy(idx_hbm.at[wid, 0], ibuf_a)

    def zero_body(i, _):
        for cc in range(D // 16):
            gbuf_a[i, pl.ds(cc * 16, 16)] = jnp.zeros((16,), jnp.float32)
        return 0

    lax.fori_loop(0, CH, zero_body, 0)
    for i in range(R_T // CH):
        pltpu.sync_copy(gbuf_a, acc_sh.at[pl.ds(s * R_T + i * CH, CH)])
    plsc.subcore_barrier()

    def gather(ibuf, m, buf, sem):
        pltpu.async_copy(x_hbm.at[ibuf.at[0, m]], buf, sem)

    def gather_wait(ibuf, m, buf, sem):
        pltpu.make_async_copy(x_hbm.at[ibuf.at[0, m]], buf, sem).wait()

    def scatter_add(ibuf, m, buf):
        pltpu.sync_copy(buf, acc_sh.at[ibuf.at[1, m]], add=True)

    for g in range(N_SUP):
        ibuf, ibuf_next = (ibuf_a, ibuf_b) if g % 2 == 0 else (ibuf_b, ibuf_a)
        if g + 1 < N_SUP:
            pltpu.async_copy(idx_hbm.at[wid, g + 1], ibuf_next, sem_i)

        gather(ibuf, 0, gbuf_a, sem_a)

        def chunk_body(k, _, ibuf=ibuf):
            m0 = 2 * k
            gather(ibuf, m0 + 1, gbuf_b, sem_b)
            gather_wait(ibuf, m0, gbuf_a, sem_a)
            scatter_add(ibuf, m0, gbuf_a)
            gather(ibuf, m0 + 2, gbuf_a, sem_a)
            gather_wait(ibuf, m0 + 1, gbuf_b, sem_b)
            scatter_add(ibuf, m0 + 1, gbuf_b)
            return 0

        lax.fori_loop(0, SUB // 2 - 1, chunk_body, 0)

        gather(ibuf, SUB - 1, gbuf_b, sem_b)
        gather_wait(ibuf, SUB - 2, gbuf_a, sem_a)
        scatter_add(ibuf, SUB - 2, gbuf_a)
        gather_wait(ibuf, SUB - 1, gbuf_b, sem_b)
        scatter_add(ibuf, SUB - 1, gbuf_b)

        if g + 1 < N_SUP:
            pltpu.make_async_copy(idx_hbm.at[wid, g + 1], ibuf_next,
                                  sem_i).wait()

    plsc.subcore_barrier()

    for i in range(R_T // CH):
        r0 = s * R_T + i * CH
        pltpu.sync_copy(acc_sh.at[pl.ds(r0, CH)], gbuf_a)
        pltpu.sync_copy(gbuf_a, out_hbm.at[c, pl.ds(r0, CH)])


_agg = functools.partial(
    pl.kernel,
    out_type=jax.ShapeDtypeStruct((NC, R_ACC, D), jnp.float32),
    mesh=plsc.VectorSubcoreMesh(core_axis_name="c", subcore_axis_name="s",
                                num_cores=NC, num_subcores=NS),
    scratch_types=[
        pltpu.VMEM((2, SUB, CH), jnp.int32),
        pltpu.VMEM((2, SUB, CH), jnp.int32),
        pltpu.VMEM((CH, D), jnp.float32),
        pltpu.VMEM((CH, D), jnp.float32),
        pltpu.VMEM_SHARED((R_ACC, D), jnp.float32),
        pltpu.SemaphoreType.DMA,
        pltpu.SemaphoreType.DMA,
        pltpu.SemaphoreType.DMA,
    ],
)(_agg_body)


def _mlp_body(x_ref, a0_ref, a1_ref, w1t_ref, b1_ref, w2t_ref, b2_ref, o_ref):
    h = x_ref[...] + a0_ref[...] + a1_ref[...]
    h = jnp.dot(h, w1t_ref[...], preferred_element_type=jnp.float32)
    h = jnp.maximum(h + b1_ref[...], 0.0)
    o_ref[...] = (jnp.dot(h, w2t_ref[...], preferred_element_type=jnp.float32)
                  + b2_ref[...])


def _mlp(x, a0, a1, w1t, b1, w2t, b2):
    blk = 2000
    grid = (N_NODES // blk,)
    row_spec = pl.BlockSpec((blk, D), lambda i: (i, 0))
    full = pl.BlockSpec((D, D), lambda i: (0, 0))
    bias = pl.BlockSpec((1, D), lambda i: (0, 0))
    return pl.pallas_call(
        _mlp_body,
        grid=grid,
        in_specs=[row_spec, row_spec, row_spec, full, bias, full, bias],
        out_specs=row_spec,
        out_shape=jax.ShapeDtypeStruct((N_NODES, D), jnp.float32),
        compiler_params=pltpu.CompilerParams(
            dimension_semantics=("arbitrary",)),
    )(x, a0, a1, w1t, b1, w2t, b2)


def kernel(x, edge_index, W1, b1, W2, b2):
    src = edge_index[0].astype(jnp.int32)
    dst = edge_index[1].astype(jnp.int32)
    n_edges = src.shape[0]
    pad = E_PAD - n_edges
    src = jnp.concatenate([src, jnp.zeros((pad,), jnp.int32)])
    dst = jnp.concatenate([dst, jnp.full((pad,), DUMMY, jnp.int32)])
    src = src.reshape(NW, N_SUP, 1, SUB, CH)
    dst = dst.reshape(NW, N_SUP, 1, SUB, CH)
    idx = jnp.concatenate([src, dst], axis=2)

    agg = _agg(x, idx)

    return _mlp(x, agg[0, :N_NODES], agg[1, :N_NODES],
                W1.T, b1.reshape(1, D), W2.T, b2.reshape(1, D))


# speedup vs baseline: 3.3194x; 1.0018x over previous
import functools

import jax
import jax.numpy as jnp
from jax import lax
from jax.experimental import pallas as pl
from jax.experimental.pallas import tpu as pltpu
from jax.experimental.pallas import tpu_sc as plsc

N_NODES = 10000
D = 128

NC = 2
NS = 16
NW = NC * NS

CH = 80
SUB = 32
N_SUP = 4
N_CH = SUB * N_SUP
E_T = N_CH * CH
E_PAD = NW * E_T
R_ACC = 10240
R_T = R_ACC // NS
DUMMY = N_NODES


def _agg_body(x_hbm, idx_hbm, out_hbm,
              ibuf_a, ibuf_b, gbuf_a, gbuf_b, acc_sh, sem_a, sem_b, sem_i):
    c = lax.axis_index("c")
    s = lax.axis_index("s")
    wid = s * NC + c

    pltpu.sync_copy(idx_hbm.at[wid, 0], ibuf_a)

    def zero_body(i, _):
        for cc in range(D // 16):
            gbuf_a[i, pl.ds(cc * 16, 16)] = jnp.zeros((16,), jnp.float32)
        return 0

    lax.fori_loop(0, CH, zero_body, 0)
    for i in range(R_T // CH):
        pltpu.sync_copy(gbuf_a, acc_sh.at[pl.ds(s * R_T + i * CH, CH)])
    plsc.subcore_barrier()

    def gather(ibuf, m, buf, sem):
        pltpu.async_copy(x_hbm.at[ibuf.at[0, m]], buf, sem)

    def gather_wait(ibuf, m, buf, sem):
        pltpu.make_async_copy(x_hbm.at[ibuf.at[0, m]], buf, sem).wait()

    def scatter_add(ibuf, m, buf):
        pltpu.sync_copy(buf, acc_sh.at[ibuf.at[1, m]], add=True)

    for g in range(N_SUP):
        ibuf, ibuf_next = (ibuf_a, ibuf_b) if g % 2 == 0 else (ibuf_b, ibuf_a)
        if g + 1 < N_SUP:
            pltpu.async_copy(idx_hbm.at[wid, g + 1], ibuf_next, sem_i)

        gather(ibuf, 0, gbuf_a, sem_a)

        def chunk_body(k, _, ibuf=ibuf):
            m0 = 2 * k
            gather(ibuf, m0 + 1, gbuf_b, sem_b)
            gather_wait(ibuf, m0, gbuf_a, sem_a)
            scatter_add(ibuf, m0, gbuf_a)
            gather(ibuf, m0 + 2, gbuf_a, sem_a)
            gather_wait(ibuf, m0 + 1, gbuf_b, sem_b)
            scatter_add(ibuf, m0 + 1, gbuf_b)
            return 0

        lax.fori_loop(0, SUB // 2 - 1, chunk_body, 0)

        gather(ibuf, SUB - 1, gbuf_b, sem_b)
        gather_wait(ibuf, SUB - 2, gbuf_a, sem_a)
        scatter_add(ibuf, SUB - 2, gbuf_a)
        gather_wait(ibuf, SUB - 1, gbuf_b, sem_b)
        scatter_add(ibuf, SUB - 1, gbuf_b)

        if g + 1 < N_SUP:
            pltpu.make_async_copy(idx_hbm.at[wid, g + 1], ibuf_next,
                                  sem_i).wait()

    plsc.subcore_barrier()

    for i in range(R_T // CH):
        r0 = s * R_T + i * CH
        pltpu.sync_copy(acc_sh.at[pl.ds(r0, CH)], gbuf_a)
        pltpu.sync_copy(gbuf_a, out_hbm.at[c, pl.ds(r0, CH)])


_agg = functools.partial(
    pl.kernel,
    out_type=jax.ShapeDtypeStruct((NC, R_ACC, D), jnp.float32),
    mesh=plsc.VectorSubcoreMesh(core_axis_name="c", subcore_axis_name="s",
                                num_cores=NC, num_subcores=NS),
    scratch_types=[
        pltpu.VMEM((2, SUB, CH), jnp.int32),
        pltpu.VMEM((2, SUB, CH), jnp.int32),
        pltpu.VMEM((CH, D), jnp.float32),
        pltpu.VMEM((CH, D), jnp.float32),
        pltpu.VMEM_SHARED((R_ACC, D), jnp.float32),
        pltpu.SemaphoreType.DMA,
        pltpu.SemaphoreType.DMA,
        pltpu.SemaphoreType.DMA,
    ],
)(_agg_body)


def _mlp_body(x_ref, a0_ref, a1_ref, w1t_ref, b1_ref, w2t_ref, b2_ref, o_ref):
    h = x_ref[...] + a0_ref[...] + a1_ref[...]
    h = jnp.dot(h, w1t_ref[...], preferred_element_type=jnp.float32)
    h = jnp.maximum(h + b1_ref[...], 0.0)
    o_ref[...] = (jnp.dot(h, w2t_ref[...], preferred_element_type=jnp.float32)
                  + b2_ref[...])


def _mlp(x, a0, a1, w1t, b1, w2t, b2):
    blk = 2000
    grid = (N_NODES // blk,)
    row_spec = pl.BlockSpec((blk, D), lambda i: (i, 0))
    full = pl.BlockSpec((D, D), lambda i: (0, 0))
    bias = pl.BlockSpec((1, D), lambda i: (0, 0))
    return pl.pallas_call(
        _mlp_body,
        grid=grid,
        in_specs=[row_spec, row_spec, row_spec, full, bias, full, bias],
        out_specs=row_spec,
        out_shape=jax.ShapeDtypeStruct((N_NODES, D), jnp.float32),
        compiler_params=pltpu.CompilerParams(
            dimension_semantics=("arbitrary",)),
    )(x, a0, a1, w1t, b1, w2t, b2)


def kernel(x, edge_index, W1, b1, W2, b2):
    src = edge_index[0].astype(jnp.int32)
    dst = edge_index[1].astype(jnp.int32)
    n_edges = src.shape[0]
    pad = E_PAD - n_edges
    dummy = DUMMY + jnp.arange(pad, dtype=jnp.int32) % (R_ACC - N_NODES)
    src = jnp.concatenate([src, jnp.zeros((pad,), jnp.int32)])
    dst = jnp.concatenate([dst, dummy])
    src = src.reshape(NW, N_SUP, 1, SUB, CH)
    dst = dst.reshape(NW, N_SUP, 1, SUB, CH)
    idx = jnp.concatenate([src, dst], axis=2)

    agg = _agg(x, idx)

    return _mlp(x, agg[0, :N_NODES], agg[1, :N_NODES],
                W1.T, b1.reshape(1, D), W2.T, b2.reshape(1, D))
